# Initial kernel scaffold; baseline (speedup 1.0000x reference)
import jax
import jax.numpy as jnp
from jax.experimental import pallas as pl


def kernel(h_p0, h_a0, paper_masks, author_masks, het_masks, p0_a_att, p0_bt2, p0_w_wi, p0_w_bt, p0_ln_g, p0_ln_b, p0_ln_list_g, p0_ln_list_b, p0_skip, p0_beta_weight, p0_overall_beta, p1_a_att, p1_bt2, p1_w_wi, p1_w_bt, p1_ln_g, p1_ln_b, p1_ln_list_g, p1_ln_list_b, p1_skip, p1_beta_weight, p1_overall_beta, p2_a_att, p2_bt2, p2_w_wi, p2_w_bt, p2_ln_g, p2_ln_b, p2_ln_list_g, p2_ln_list_b, p2_skip, p2_beta_weight, p2_overall_beta, p3_a_att, p3_bt2, p3_w_wi, p3_w_bt, p3_ln_g, p3_ln_b, p3_ln_list_g, p3_ln_list_b, p3_skip, p3_beta_weight, p3_overall_beta, q0_a_att, q0_bt2, q0_w_bt, q0_ln_g, q0_ln_b, q0_skip, q1_a_att, q1_bt2, q1_w_bt, q1_ln_g, q1_ln_b, q1_skip, q2_a_att, q2_bt2, q2_w_bt, q2_ln_g, q2_ln_b, q2_skip, q3_a_att, q3_bt2, q3_w_bt, q3_ln_g, q3_ln_b, q3_skip, top_ln_g, top_ln_b):
    raise NotImplementedError("write your pallas kernel here")


# speedup vs baseline: 1.4038x; 1.4038x over previous
import jax
import jax.numpy as jnp
from jax.experimental import pallas as pl
from jax.experimental.pallas import tpu as pltpu
from functools import partial

NEG_SLOPE = 0.2
LN_EPS = 1e-5
H = 128
TD = 256
VMEM_LIMIT = 48 * 1024 * 1024
GELU_C = 0.7978845608028654


def _ln(y, g, b):
    mu = jnp.sum(y, axis=-1, keepdims=True) * (1.0 / H)
    d = y - mu
    var = jnp.sum(d * d, axis=-1, keepdims=True) * (1.0 / H)
    return d * jax.lax.rsqrt(var + LN_EPS) * g + b


def _xform_kernel(x_ref, w_ref, aj_ref, xt_ref, lj_ref, cm_ref):
    xt = jnp.dot(x_ref[...], w_ref[...],
                 preferred_element_type=jnp.float32).astype(jnp.bfloat16)
    xt_ref[...] = xt
    lj = jnp.dot(xt, aj_ref[...], preferred_element_type=jnp.float32)
    lj_ref[...] = lj
    cm_ref[...] = jnp.broadcast_to(jnp.max(lj, axis=0, keepdims=True), (8, H))


def _xform(x, w_wi, ajw):
    n = x.shape[0]
    return pl.pallas_call(
        _xform_kernel,
        out_shape=(jax.ShapeDtypeStruct((n, H), jnp.bfloat16),
                   jax.ShapeDtypeStruct((n, H), jnp.float32),
                   jax.ShapeDtypeStruct((8, H), jnp.float32)),
        grid_spec=pltpu.PrefetchScalarGridSpec(
            num_scalar_prefetch=0,
            grid=(1,),
            in_specs=[pl.BlockSpec((n, H), lambda r: (0, 0)),
                      pl.BlockSpec((H, H), lambda r: (0, 0)),
                      pl.BlockSpec((H, H), lambda r: (0, 0))],
            out_specs=(pl.BlockSpec((n, H), lambda r: (0, 0)),
                       pl.BlockSpec((n, H), lambda r: (0, 0)),
                       pl.BlockSpec((8, H), lambda r: (0, 0))),
        ),
        compiler_params=pltpu.CompilerParams(
            dimension_semantics=("arbitrary",),
            vmem_limit_bytes=VMEM_LIMIT),
    )(x, w_wi, ajw)


def _dgat_kernel(xs_ref, xd_ref, lj_ref, cm_ref, mask_ref, lng_ref, lnb_ref,
                 wbt_ref, bt2_ref, gw_ref, sc_ref, g1_ref, b1_ref, o_ref,
                 *, num_types):
    xs = xs_ref[...]
    xsf = xs.astype(jnp.float32)
    c = jnp.max(cm_ref[0])
    wbt = wbt_ref[...]
    x = xd_ref[...].astype(jnp.float32)
    base = jnp.sum(x * bt2_ref[0:1, :], axis=-1, keepdims=True)

    outs, rs = [], []
    for t in range(num_types):
        w = jnp.exp(lj_ref[:, t:t + 1] - c)
        u = (w * xsf).astype(jnp.bfloat16)
        mb = mask_ref[t].astype(jnp.bfloat16)
        num = jnp.dot(mb, u, preferred_element_type=jnp.float32)
        den = jax.lax.dot_general(mb, w.astype(jnp.bfloat16),
                                  (((1,), (0,)), ((), ())),
                                  preferred_element_type=jnp.float32)
        agg = num * pl.reciprocal(den + 1e-30, approx=True)
        agg = _ln(agg, lng_ref[t], lnb_ref[t])
        hn = jnp.where(agg > 0, agg, NEG_SLOPE * agg)
        o = jnp.dot(hn.astype(jnp.bfloat16), wbt,
                    preferred_element_type=jnp.float32)
        o = o.astype(jnp.bfloat16).astype(jnp.float32)
        outs.append(o)
        rs.append(base + jnp.sum(o * bt2_ref[1:2, :], axis=-1, keepdims=True))

    mx = rs[0]
    for t in range(1, num_types):
        mx = jnp.maximum(mx, rs[t])
    es = [jnp.exp(r - mx) for r in rs]
    denom = es[0]
    for t in range(1, num_types):
        denom = denom + es[t]
    inv = pl.reciprocal(denom, approx=True)
    beta_scale = sc_ref[0]
    fw = sc_ref[1]
    res = jnp.zeros_like(x)
    for t in range(num_types):
        res = res + outs[t] * (beta_scale * (es[t] * inv) + gw_ref[t])
    gel = 0.5 * res * (1.0 + jnp.tanh(GELU_C * (res + 0.044715 * res * res * res)))
    y = gel * fw + x * (1.0 - fw)
    y = _ln(y, g1_ref[...], b1_ref[...])
    o_ref[...] = y.astype(o_ref.dtype)


def _dgat_conv(x, masks, ajw, p):
    num_types, nd, ns = masks.shape
    xt, lj, cm = _xform(x, p["w_wi"], ajw)
    grid = (nd // TD,)
    return pl.pallas_call(
        partial(_dgat_kernel, num_types=num_types),
        out_shape=jax.ShapeDtypeStruct((nd, H), jnp.bfloat16),
        grid_spec=pltpu.PrefetchScalarGridSpec(
            num_scalar_prefetch=0,
            grid=grid,
            in_specs=[
                pl.BlockSpec((ns, H), lambda d: (0, 0)),
                pl.BlockSpec((TD, H), lambda d: (d, 0)),
                pl.BlockSpec((ns, H), lambda d: (0, 0)),
                pl.BlockSpec((8, H), lambda d: (0, 0)),
                pl.BlockSpec((num_types, TD, ns), lambda d: (0, d, 0)),
                pl.BlockSpec((num_types, 1, H), lambda d: (0, 0, 0)),
                pl.BlockSpec((num_types, 1, H), lambda d: (0, 0, 0)),
                pl.BlockSpec((H, H), lambda d: (0, 0)),
                pl.BlockSpec((2, H), lambda d: (0, 0)),
                pl.BlockSpec(memory_space=pltpu.MemorySpace.SMEM),
                pl.BlockSpec(memory_space=pltpu.MemorySpace.SMEM),
                pl.BlockSpec((1, H), lambda d: (0, 0)),
                pl.BlockSpec((1, H), lambda d: (0, 0)),
            ],
            out_specs=pl.BlockSpec((TD, H), lambda d: (d, 0)),
        ),
        compiler_params=pltpu.CompilerParams(
            dimension_semantics=("parallel",),
            vmem_limit_bytes=VMEM_LIMIT),
    )(xt, xt, lj, cm, masks, p["ln_list_g"], p["ln_list_b"],
      p["w_bt"], p["bt2"], p["gw"], p["sc"], p["ln_g"], p["ln_b"])


def _het_kernel(xs_ref, xd_ref, aj_ref, mask_ref, wbt_ref, bt2_ref,
                gw_ref, sc_ref, g1_ref, b1_ref, *rest,
                num_types, trans, extra_ln):
    if extra_ln:
        g2_ref, b2_ref, o_ref = rest
    else:
        (o_ref,) = rest

    xs = xs_ref[...]
    xsf = xs.astype(jnp.float32)
    lj = jnp.dot(xs, aj_ref[...], preferred_element_type=jnp.float32)
    c = jnp.max(lj)
    wbt = wbt_ref[...]
    x = xd_ref[...].astype(jnp.float32)
    base = jnp.sum(x * bt2_ref[0:1, :], axis=-1, keepdims=True)

    outs, rs = [], []
    for t in range(num_types):
        w = jnp.exp(lj[:, t:t + 1] - c)
        u = (w * xsf).astype(jnp.bfloat16)
        mb = mask_ref[t].astype(jnp.bfloat16)
        cdim = 1 - int(trans)
        num = jax.lax.dot_general(mb, u, (((cdim,), (0,)), ((), ())),
                                  preferred_element_type=jnp.float32)
        den = jax.lax.dot_general(mb, w.astype(jnp.bfloat16),
                                  (((cdim,), (0,)), ((), ())),
                                  preferred_element_type=jnp.float32)
        agg = num * pl.reciprocal(den + 1e-30, approx=True)
        hn = jnp.where(agg > 0, agg, NEG_SLOPE * agg)
        o = jnp.dot(hn.astype(jnp.bfloat16), wbt,
                    preferred_element_type=jnp.float32)
        o = o.astype(jnp.bfloat16).astype(jnp.float32)
        outs.append(o)
        rs.append(base + jnp.sum(o * bt2_ref[1:2, :], axis=-1, keepdims=True))

    mx = rs[0]
    for t in range(1, num_types):
        mx = jnp.maximum(mx, rs[t])
    es = [jnp.exp(r - mx) for r in rs]
    denom = es[0]
    for t in range(1, num_types):
        denom = denom + es[t]
    inv = pl.reciprocal(denom, approx=True)
    beta_scale = sc_ref[0]
    fw = sc_ref[1]
    res = jnp.zeros_like(x)
    for t in range(num_types):
        res = res + outs[t] * (beta_scale * (es[t] * inv) + gw_ref[t])
    gel = 0.5 * res * (1.0 + jnp.tanh(GELU_C * (res + 0.044715 * res * res * res)))
    y = gel * fw + x * (1.0 - fw)
    y = _ln(y, g1_ref[...], b1_ref[...])
    if extra_ln:
        y = _ln(y, g2_ref[...], b2_ref[...])
    o_ref[...] = y.astype(o_ref.dtype)


def _het_conv(src_hid, dst_hid, masks, ajw, p, *, trans, extra_ln=None):
    num_types = masks.shape[0]
    nd = dst_hid.shape[0]
    ns = src_hid.shape[0]
    if trans:
        mask_spec = pl.BlockSpec((num_types, ns, TD), lambda d: (0, 0, d))
    else:
        mask_spec = pl.BlockSpec((num_types, TD, ns), lambda d: (0, d, 0))
    in_specs = [
        pl.BlockSpec((ns, H), lambda d: (0, 0)),
        pl.BlockSpec((TD, H), lambda d: (d, 0)),
        pl.BlockSpec((H, H), lambda d: (0, 0)),
        mask_spec,
        pl.BlockSpec((H, H), lambda d: (0, 0)),
        pl.BlockSpec((2, H), lambda d: (0, 0)),
        pl.BlockSpec(memory_space=pltpu.MemorySpace.SMEM),
        pl.BlockSpec(memory_space=pltpu.MemorySpace.SMEM),
        pl.BlockSpec((1, H), lambda d: (0, 0)),
        pl.BlockSpec((1, H), lambda d: (0, 0)),
    ]
    args = [src_hid, dst_hid, ajw, masks, p["w_bt"], p["bt2"],
            p["gw"], p["sc"], p["ln_g"], p["ln_b"]]
    if extra_ln is not None:
        in_specs += [pl.BlockSpec((1, H), lambda d: (0, 0)),
                     pl.BlockSpec((1, H), lambda d: (0, 0))]
        args += [extra_ln[0], extra_ln[1]]
    out_dtype = jnp.float32 if extra_ln is not None else jnp.bfloat16
    return pl.pallas_call(
        partial(_het_kernel, num_types=num_types, trans=trans,
                extra_ln=extra_ln is not None),
        out_shape=jax.ShapeDtypeStruct((nd, H), out_dtype),
        grid_spec=pltpu.PrefetchScalarGridSpec(
            num_scalar_prefetch=0,
            grid=(nd // TD,),
            in_specs=in_specs,
            out_specs=pl.BlockSpec((TD, H), lambda d: (d, 0)),
        ),
        compiler_params=pltpu.CompilerParams(
            dimension_semantics=("parallel",),
            vmem_limit_bytes=VMEM_LIMIT),
    )(*args)


def _aj_matrix(a_att):
    a_j = a_att[:, 1, :]
    t = a_j.shape[0]
    return jnp.pad(a_j.T.astype(jnp.bfloat16), ((0, 0), (0, H - t)))


def _dgat_scalars(p):
    bw = jax.nn.sigmoid(p["beta_weight"])[0]
    gw = bw * jax.nn.softmax(p["overall_beta"])
    fw = jax.nn.sigmoid(p["skip"])[0]
    sc = jnp.stack([1.0 - bw, fw])
    return gw.astype(jnp.float32), sc.astype(jnp.float32)


def _het_scalars(p):
    gw = jnp.zeros((2,), jnp.float32)
    sc = jnp.concatenate([jnp.ones((1,), jnp.float32),
                          jax.nn.sigmoid(p["skip"])]).astype(jnp.float32)
    return gw, sc


def kernel(h_p0, h_a0, paper_masks, author_masks, het_masks, p0_a_att, p0_bt2, p0_w_wi, p0_w_bt, p0_ln_g, p0_ln_b, p0_ln_list_g, p0_ln_list_b, p0_skip, p0_beta_weight, p0_overall_beta, p1_a_att, p1_bt2, p1_w_wi, p1_w_bt, p1_ln_g, p1_ln_b, p1_ln_list_g, p1_ln_list_b, p1_skip, p1_beta_weight, p1_overall_beta, p2_a_att, p2_bt2, p2_w_wi, p2_w_bt, p2_ln_g, p2_ln_b, p2_ln_list_g, p2_ln_list_b, p2_skip, p2_beta_weight, p2_overall_beta, p3_a_att, p3_bt2, p3_w_wi, p3_w_bt, p3_ln_g, p3_ln_b, p3_ln_list_g, p3_ln_list_b, p3_skip, p3_beta_weight, p3_overall_beta, q0_a_att, q0_bt2, q0_w_bt, q0_ln_g, q0_ln_b, q0_skip, q1_a_att, q1_bt2, q1_w_bt, q1_ln_g, q1_ln_b, q1_skip, q2_a_att, q2_bt2, q2_w_bt, q2_ln_g, q2_ln_b, q2_skip, q3_a_att, q3_bt2, q3_w_bt, q3_ln_g, q3_ln_b, q3_skip, top_ln_g, top_ln_b):
    hgt = [
        dict(a_att=p0_a_att, bt2=p0_bt2, w_wi=p0_w_wi, w_bt=p0_w_bt,
             ln_g=p0_ln_g, ln_b=p0_ln_b, ln_list_g=p0_ln_list_g,
             ln_list_b=p0_ln_list_b, skip=p0_skip, beta_weight=p0_beta_weight,
             overall_beta=p0_overall_beta),
        dict(a_att=p1_a_att, bt2=p1_bt2, w_wi=p1_w_wi, w_bt=p1_w_bt,
             ln_g=p1_ln_g, ln_b=p1_ln_b, ln_list_g=p1_ln_list_g,
             ln_list_b=p1_ln_list_b, skip=p1_skip, beta_weight=p1_beta_weight,
             overall_beta=p1_overall_beta),
        dict(a_att=p2_a_att, bt2=p2_bt2, w_wi=p2_w_wi, w_bt=p2_w_bt,
             ln_g=p2_ln_g, ln_b=p2_ln_b, ln_list_g=p2_ln_list_g,
             ln_list_b=p2_ln_list_b, skip=p2_skip, beta_weight=p2_beta_weight,
             overall_beta=p2_overall_beta),
        dict(a_att=p3_a_att, bt2=p3_bt2, w_wi=p3_w_wi, w_bt=p3_w_bt,
             ln_g=p3_ln_g, ln_b=p3_ln_b, ln_list_g=p3_ln_list_g,
             ln_list_b=p3_ln_list_b, skip=p3_skip, beta_weight=p3_beta_weight,
             overall_beta=p3_overall_beta),
    ]
    het = [
        dict(a_att=q0_a_att, bt2=q0_bt2, w_bt=q0_w_bt, ln_g=q0_ln_g,
             ln_b=q0_ln_b, skip=q0_skip),
        dict(a_att=q1_a_att, bt2=q1_bt2, w_bt=q1_w_bt, ln_g=q1_ln_g,
             ln_b=q1_ln_b, skip=q1_skip),
        dict(a_att=q2_a_att, bt2=q2_bt2, w_bt=q2_w_bt, ln_g=q2_ln_g,
             ln_b=q2_ln_b, skip=q2_skip),
        dict(a_att=q3_a_att, bt2=q3_bt2, w_bt=q3_w_bt, ln_g=q3_ln_g,
             ln_b=q3_ln_b, skip=q3_skip),
    ]
    for p in hgt:
        p["gw"], p["sc"] = _dgat_scalars(p)
        p["ajw"] = _aj_matrix(p["a_att"])
    for p in het:
        p["gw"], p["sc"] = _het_scalars(p)
        p["ajw"] = _aj_matrix(p["a_att"])

    h_p = h_p0.astype(jnp.bfloat16)
    h_a = h_a0.astype(jnp.bfloat16)
    for hl in range(2):
        h_p = _dgat_conv(h_p, paper_masks, hgt[2 * hl]["ajw"], hgt[2 * hl])
        h_a = _dgat_conv(h_a, author_masks, hgt[2 * hl + 1]["ajw"],
                         hgt[2 * hl + 1])
    final_ln = (top_ln_g, top_ln_b)
    for ly in range(2):
        extra = final_ln if ly == 1 else None
        p_hid = _het_conv(h_a, h_p, het_masks, het[2 * ly]["ajw"],
                          het[2 * ly], trans=False, extra_ln=extra)
        a_hid = _het_conv(h_p, h_a, het_masks, het[2 * ly + 1]["ajw"],
                          het[2 * ly + 1], trans=True, extra_ln=extra)
        h_a, h_p = a_hid, p_hid
    return h_a, h_p


# speedup vs baseline: 1.7442x; 1.2424x over previous
import jax
import jax.numpy as jnp
from jax.experimental import pallas as pl
from jax.experimental.pallas import tpu as pltpu
from functools import partial

NEG_SLOPE = 0.2
LN_EPS = 1e-5
H = 128
H2 = 256
TD = 256
VMEM_LIMIT = 48 * 1024 * 1024
GELU_C = 0.7978845608028654


def _ln(y, g, b):
    mu = jnp.sum(y, axis=-1, keepdims=True) * (1.0 / H)
    d = y - mu
    var = jnp.sum(d * d, axis=-1, keepdims=True) * (1.0 / H)
    return d * jax.lax.rsqrt(var + LN_EPS) * g + b


def _aug_sources(xsf, lj, num_types):
    c = jnp.max(lj)
    u2 = []
    for t in range(num_types):
        w = jnp.exp(lj[:, t:t + 1] - c)
        u = (w * xsf).astype(jnp.bfloat16)
        wp = jnp.pad(w, ((0, 0), (0, H - 1))).astype(jnp.bfloat16)
        u2.append(jnp.concatenate([u, wp], axis=1))
    return u2


def _combine_tail(outs, rs, x, gw_ref, sc_ref, num_types):
    mx = rs[0]
    for t in range(1, num_types):
        mx = jnp.maximum(mx, rs[t])
    es = [jnp.exp(r - mx) for r in rs]
    denom = es[0]
    for t in range(1, num_types):
        denom = denom + es[t]
    inv = pl.reciprocal(denom, approx=True)
    beta_scale = sc_ref[0]
    fw = sc_ref[1]
    res = jnp.zeros_like(x)
    for t in range(num_types):
        res = res + outs[t] * (beta_scale * (es[t] * inv) + gw_ref[t])
    gel = 0.5 * res * (1.0 + jnp.tanh(GELU_C * (res + 0.044715 * res * res * res)))
    return gel * fw + x * (1.0 - fw)


def _xform_kernel(x_ref, w_ref, aj_ref, xt_ref, u_ref, *, num_types):
    xt = jnp.dot(x_ref[...], w_ref[...],
                 preferred_element_type=jnp.float32).astype(jnp.bfloat16)
    xt_ref[...] = xt
    lj = jnp.dot(xt, aj_ref[...], preferred_element_type=jnp.float32)
    u2 = _aug_sources(xt.astype(jnp.float32), lj, num_types)
    for t in range(num_types):
        u_ref[t] = u2[t]


def _xform(x, w_wi, ajw, num_types):
    n = x.shape[0]
    return pl.pallas_call(
        partial(_xform_kernel, num_types=num_types),
        out_shape=(jax.ShapeDtypeStruct((n, H), jnp.bfloat16),
                   jax.ShapeDtypeStruct((num_types, n, H2), jnp.bfloat16)),
        grid_spec=pltpu.PrefetchScalarGridSpec(
            num_scalar_prefetch=0,
            grid=(1,),
            in_specs=[pl.BlockSpec((n, H), lambda r: (0, 0)),
                      pl.BlockSpec((H, H), lambda r: (0, 0)),
                      pl.BlockSpec((H, H), lambda r: (0, 0))],
            out_specs=(pl.BlockSpec((n, H), lambda r: (0, 0)),
                       pl.BlockSpec((num_types, n, H2), lambda r: (0, 0, 0))),
        ),
        compiler_params=pltpu.CompilerParams(
            dimension_semantics=("arbitrary",),
            vmem_limit_bytes=VMEM_LIMIT),
    )(x, w_wi, ajw)


def _dgat_kernel(xd_ref, u_ref, mask_ref, lng_ref, lnb_ref,
                 wbt_ref, bt2_ref, gw_ref, sc_ref, g1_ref, b1_ref, o_ref,
                 *, num_types):
    wbt = wbt_ref[...]
    x = xd_ref[...].astype(jnp.float32)
    base = jnp.sum(x * bt2_ref[0:1, :], axis=-1, keepdims=True)

    outs, rs = [], []
    for t in range(num_types):
        mb = mask_ref[t].astype(jnp.bfloat16)
        num2 = jnp.dot(mb, u_ref[t], preferred_element_type=jnp.float32)
        num = num2[:, :H]
        den = num2[:, H:H + 1]
        agg = num * pl.reciprocal(den + 1e-30, approx=True)
        agg = _ln(agg, lng_ref[t], lnb_ref[t])
        hn = jnp.where(agg > 0, agg, NEG_SLOPE * agg)
        o = jnp.dot(hn.astype(jnp.bfloat16), wbt,
                    preferred_element_type=jnp.float32)
        o = o.astype(jnp.bfloat16).astype(jnp.float32)
        outs.append(o)
        rs.append(base + jnp.sum(o * bt2_ref[1:2, :], axis=-1, keepdims=True))

    y = _combine_tail(outs, rs, x, gw_ref, sc_ref, num_types)
    y = _ln(y, g1_ref[...], b1_ref[...])
    o_ref[...] = y.astype(o_ref.dtype)


def _dgat_conv(x, masks, ajw, p):
    num_types, nd, ns = masks.shape
    xt, u2 = _xform(x, p["w_wi"], ajw, num_types)
    return pl.pallas_call(
        partial(_dgat_kernel, num_types=num_types),
        out_shape=jax.ShapeDtypeStruct((nd, H), jnp.bfloat16),
        grid_spec=pltpu.PrefetchScalarGridSpec(
            num_scalar_prefetch=0,
            grid=(nd // TD,),
            in_specs=[
                pl.BlockSpec((TD, H), lambda d: (d, 0)),
                pl.BlockSpec((num_types, ns, H2), lambda d: (0, 0, 0)),
                pl.BlockSpec((num_types, TD, ns), lambda d: (0, d, 0)),
                pl.BlockSpec((num_types, 1, H), lambda d: (0, 0, 0)),
                pl.BlockSpec((num_types, 1, H), lambda d: (0, 0, 0)),
                pl.BlockSpec((H, H), lambda d: (0, 0)),
                pl.BlockSpec((2, H), lambda d: (0, 0)),
                pl.BlockSpec(memory_space=pltpu.MemorySpace.SMEM),
                pl.BlockSpec(memory_space=pltpu.MemorySpace.SMEM),
                pl.BlockSpec((1, H), lambda d: (0, 0)),
                pl.BlockSpec((1, H), lambda d: (0, 0)),
            ],
            out_specs=pl.BlockSpec((TD, H), lambda d: (d, 0)),
        ),
        compiler_params=pltpu.CompilerParams(
            dimension_semantics=("arbitrary",),
            vmem_limit_bytes=VMEM_LIMIT),
    )(xt, u2, masks, p["ln_list_g"], p["ln_list_b"],
      p["w_bt"], p["bt2"], p["gw"], p["sc"], p["ln_g"], p["ln_b"])


def _het_kernel(xs_ref, xd_ref, aj_ref, mask_ref, wbt_ref, bt2_ref,
                gw_ref, sc_ref, g1_ref, b1_ref, *rest,
                num_types, trans, extra_ln):
    if extra_ln:
        g2_ref, b2_ref, o_ref, u_sc = rest
    else:
        o_ref, u_sc = rest

    @pl.when(pl.program_id(0) == 0)
    def _prep():
        xs = xs_ref[...]
        lj = jnp.dot(xs, aj_ref[...], preferred_element_type=jnp.float32)
        u2 = _aug_sources(xs.astype(jnp.float32), lj, num_types)
        for t in range(num_types):
            u_sc[t] = u2[t]

    wbt = wbt_ref[...]
    x = xd_ref[...].astype(jnp.float32)
    base = jnp.sum(x * bt2_ref[0:1, :], axis=-1, keepdims=True)

    outs, rs = [], []
    for t in range(num_types):
        mb = mask_ref[t].astype(jnp.bfloat16)
        cdim = 1 - int(trans)
        num2 = jax.lax.dot_general(mb, u_sc[t], (((cdim,), (0,)), ((), ())),
                                   preferred_element_type=jnp.float32)
        num = num2[:, :H]
        den = num2[:, H:H + 1]
        agg = num * pl.reciprocal(den + 1e-30, approx=True)
        hn = jnp.where(agg > 0, agg, NEG_SLOPE * agg)
        o = jnp.dot(hn.astype(jnp.bfloat16), wbt,
                    preferred_element_type=jnp.float32)
        o = o.astype(jnp.bfloat16).astype(jnp.float32)
        outs.append(o)
        rs.append(base + jnp.sum(o * bt2_ref[1:2, :], axis=-1, keepdims=True))

    y = _combine_tail(outs, rs, x, gw_ref, sc_ref, num_types)
    y = _ln(y, g1_ref[...], b1_ref[...])
    if extra_ln:
        y = _ln(y, g2_ref[...], b2_ref[...])
    o_ref[...] = y.astype(o_ref.dtype)


def _het_conv(src_hid, dst_hid, masks, ajw, p, *, trans, extra_ln=None):
    num_types = masks.shape[0]
    nd = dst_hid.shape[0]
    ns = src_hid.shape[0]
    if trans:
        mask_spec = pl.BlockSpec((num_types, ns, TD), lambda d: (0, 0, d))
    else:
        mask_spec = pl.BlockSpec((num_types, TD, ns), lambda d: (0, d, 0))
    in_specs = [
        pl.BlockSpec((ns, H), lambda d: (0, 0)),
        pl.BlockSpec((TD, H), lambda d: (d, 0)),
        pl.BlockSpec((H, H), lambda d: (0, 0)),
        mask_spec,
        pl.BlockSpec((H, H), lambda d: (0, 0)),
        pl.BlockSpec((2, H), lambda d: (0, 0)),
        pl.BlockSpec(memory_space=pltpu.MemorySpace.SMEM),
        pl.BlockSpec(memory_space=pltpu.MemorySpace.SMEM),
        pl.BlockSpec((1, H), lambda d: (0, 0)),
        pl.BlockSpec((1, H), lambda d: (0, 0)),
    ]
    args = [src_hid, dst_hid, ajw, masks, p["w_bt"], p["bt2"],
            p["gw"], p["sc"], p["ln_g"], p["ln_b"]]
    if extra_ln is not None:
        in_specs += [pl.BlockSpec((1, H), lambda d: (0, 0)),
                     pl.BlockSpec((1, H), lambda d: (0, 0))]
        args += [extra_ln[0], extra_ln[1]]
    out_dtype = jnp.float32 if extra_ln is not None else jnp.bfloat16
    return pl.pallas_call(
        partial(_het_kernel, num_types=num_types, trans=trans,
                extra_ln=extra_ln is not None),
        out_shape=jax.ShapeDtypeStruct((nd, H), out_dtype),
        grid_spec=pltpu.PrefetchScalarGridSpec(
            num_scalar_prefetch=0,
            grid=(nd // TD,),
            in_specs=in_specs,
            out_specs=pl.BlockSpec((TD, H), lambda d: (d, 0)),
            scratch_shapes=[pltpu.VMEM((num_types, ns, H2), jnp.bfloat16)],
        ),
        compiler_params=pltpu.CompilerParams(
            dimension_semantics=("arbitrary",),
            vmem_limit_bytes=VMEM_LIMIT),
    )(*args)


def _aj_matrix(a_att):
    a_j = a_att[:, 1, :]
    t = a_j.shape[0]
    return jnp.pad(a_j.T.astype(jnp.bfloat16), ((0, 0), (0, H - t)))


def _dgat_scalars(p):
    bw = jax.nn.sigmoid(p["beta_weight"])[0]
    gw = bw * jax.nn.softmax(p["overall_beta"])
    fw = jax.nn.sigmoid(p["skip"])[0]
    sc = jnp.stack([1.0 - bw, fw])
    return gw.astype(jnp.float32), sc.astype(jnp.float32)


def _het_scalars(p):
    gw = jnp.zeros((2,), jnp.float32)
    sc = jnp.concatenate([jnp.ones((1,), jnp.float32),
                          jax.nn.sigmoid(p["skip"])]).astype(jnp.float32)
    return gw, sc


def kernel(h_p0, h_a0, paper_masks, author_masks, het_masks, p0_a_att, p0_bt2, p0_w_wi, p0_w_bt, p0_ln_g, p0_ln_b, p0_ln_list_g, p0_ln_list_b, p0_skip, p0_beta_weight, p0_overall_beta, p1_a_att, p1_bt2, p1_w_wi, p1_w_bt, p1_ln_g, p1_ln_b, p1_ln_list_g, p1_ln_list_b, p1_skip, p1_beta_weight, p1_overall_beta, p2_a_att, p2_bt2, p2_w_wi, p2_w_bt, p2_ln_g, p2_ln_b, p2_ln_list_g, p2_ln_list_b, p2_skip, p2_beta_weight, p2_overall_beta, p3_a_att, p3_bt2, p3_w_wi, p3_w_bt, p3_ln_g, p3_ln_b, p3_ln_list_g, p3_ln_list_b, p3_skip, p3_beta_weight, p3_overall_beta, q0_a_att, q0_bt2, q0_w_bt, q0_ln_g, q0_ln_b, q0_skip, q1_a_att, q1_bt2, q1_w_bt, q1_ln_g, q1_ln_b, q1_skip, q2_a_att, q2_bt2, q2_w_bt, q2_ln_g, q2_ln_b, q2_skip, q3_a_att, q3_bt2, q3_w_bt, q3_ln_g, q3_ln_b, q3_skip, top_ln_g, top_ln_b):
    hgt = [
        dict(a_att=p0_a_att, bt2=p0_bt2, w_wi=p0_w_wi, w_bt=p0_w_bt,
             ln_g=p0_ln_g, ln_b=p0_ln_b, ln_list_g=p0_ln_list_g,
             ln_list_b=p0_ln_list_b, skip=p0_skip, beta_weight=p0_beta_weight,
             overall_beta=p0_overall_beta),
        dict(a_att=p1_a_att, bt2=p1_bt2, w_wi=p1_w_wi, w_bt=p1_w_bt,
             ln_g=p1_ln_g, ln_b=p1_ln_b, ln_list_g=p1_ln_list_g,
             ln_list_b=p1_ln_list_b, skip=p1_skip, beta_weight=p1_beta_weight,
             overall_beta=p1_overall_beta),
        dict(a_att=p2_a_att, bt2=p2_bt2, w_wi=p2_w_wi, w_bt=p2_w_bt,
             ln_g=p2_ln_g, ln_b=p2_ln_b, ln_list_g=p2_ln_list_g,
             ln_list_b=p2_ln_list_b, skip=p2_skip, beta_weight=p2_beta_weight,
             overall_beta=p2_overall_beta),
        dict(a_att=p3_a_att, bt2=p3_bt2, w_wi=p3_w_wi, w_bt=p3_w_bt,
             ln_g=p3_ln_g, ln_b=p3_ln_b, ln_list_g=p3_ln_list_g,
             ln_list_b=p3_ln_list_b, skip=p3_skip, beta_weight=p3_beta_weight,
             overall_beta=p3_overall_beta),
    ]
    het = [
        dict(a_att=q0_a_att, bt2=q0_bt2, w_bt=q0_w_bt, ln_g=q0_ln_g,
             ln_b=q0_ln_b, skip=q0_skip),
        dict(a_att=q1_a_att, bt2=q1_bt2, w_bt=q1_w_bt, ln_g=q1_ln_g,
             ln_b=q1_ln_b, skip=q1_skip),
        dict(a_att=q2_a_att, bt2=q2_bt2, w_bt=q2_w_bt, ln_g=q2_ln_g,
             ln_b=q2_ln_b, skip=q2_skip),
        dict(a_att=q3_a_att, bt2=q3_bt2, w_bt=q3_w_bt, ln_g=q3_ln_g,
             ln_b=q3_ln_b, skip=q3_skip),
    ]
    for p in hgt:
        p["gw"], p["sc"] = _dgat_scalars(p)
        p["ajw"] = _aj_matrix(p["a_att"])
    for p in het:
        p["gw"], p["sc"] = _het_scalars(p)
        p["ajw"] = _aj_matrix(p["a_att"])

    h_p = h_p0.astype(jnp.bfloat16)
    h_a = h_a0.astype(jnp.bfloat16)
    for hl in range(2):
        h_p = _dgat_conv(h_p, paper_masks, hgt[2 * hl]["ajw"], hgt[2 * hl])
        h_a = _dgat_conv(h_a, author_masks, hgt[2 * hl + 1]["ajw"],
                         hgt[2 * hl + 1])
    final_ln = (top_ln_g, top_ln_b)
    for ly in range(2):
        extra = final_ln if ly == 1 else None
        p_hid = _het_conv(h_a, h_p, het_masks, het[2 * ly]["ajw"],
                          het[2 * ly], trans=False, extra_ln=extra)
        a_hid = _het_conv(h_p, h_a, het_masks, het[2 * ly + 1]["ajw"],
                          het[2 * ly + 1], trans=True, extra_ln=extra)
        h_a, h_p = a_hid, p_hid
    return h_a, h_p
